# Initial kernel scaffold; baseline (speedup 1.0000x reference)
import jax
import jax.numpy as jnp
from jax.experimental import pallas as pl


def kernel(x_in, edge_index, idx, W1, b1, W2, b2, W3, b3, W4, b4):
    raise NotImplementedError("write your pallas kernel here")


# speedup vs baseline: 3.0370x; 3.0370x over previous
import functools

import jax
import jax.numpy as jnp
from jax import lax
from jax.experimental import pallas as pl
from jax.experimental.pallas import tpu as pltpu
from jax.experimental.pallas import tpu_sc as plsc

N_NODES = 10000
N_EDGES = 160000
D = 256
HALF = 128
N_GRAPHS = 64
N_CLASS = 64

NC = 2
NS = 16
CH = 128
CHUNKS_PER_TILE = -(-N_EDGES // (NS * CH))
E_PAD = NS * CHUNKS_PER_TILE * CH
ROWS_PER_TILE = N_NODES // NS
ACC_ROWS = N_NODES + NS

_PREC = lax.Precision.HIGHEST


def _fc_in_body(x_ref, wt_ref, b_ref, out_ref):
    y = lax.dot_general(x_ref[...], wt_ref[...], (((1,), (0,)), ((), ())),
                        precision=_PREC, preferred_element_type=jnp.float32)
    y = y + b_ref[...]
    out_ref[0] = y[:, :HALF]
    out_ref[1] = y[:, HALF:]


def _fc_in(x, wt, b2d, relu_halves):
    blk = 1000
    grid = (N_NODES // blk,)
    if relu_halves:
        in_specs = [pl.BlockSpec((2, blk, HALF), lambda i: (0, i, 0))]
        body = _fc_mid_body
    else:
        in_specs = [pl.BlockSpec((blk, D), lambda i: (i, 0))]
        body = _fc_in_body
    in_specs += [
        pl.BlockSpec((D, D), lambda i: (0, 0)),
        pl.BlockSpec((1, D), lambda i: (0, 0)),
    ]
    return pl.pallas_call(
        body,
        grid=grid,
        in_specs=in_specs,
        out_specs=pl.BlockSpec((2, blk, HALF), lambda i: (0, i, 0)),
        out_shape=jax.ShapeDtypeStruct((2, N_NODES, HALF), jnp.float32),
    )(x, wt, b2d)


def _fc_mid_body(p_ref, wt_ref, b_ref, out_ref):
    h = jnp.concatenate([p_ref[0], p_ref[1]], axis=-1)
    h = jnp.maximum(h, 0.0)
    y = lax.dot_general(h, wt_ref[...], (((1,), (0,)), ((), ())),
                        precision=_PREC, preferred_element_type=jnp.float32)
    y = y + b_ref[...]
    out_ref[0] = y[:, :HALF]
    out_ref[1] = y[:, HALF:]


def _prop_body(hsrc, src2, dst2, out, acc_sh, src_v, dst_v, rows_v, sem):
    c = lax.axis_index("c")
    s = lax.axis_index("s")

    r0 = s * ROWS_PER_TILE
    pltpu.sync_copy(hsrc.at[pl.ds(c * N_NODES + r0, ROWS_PER_TILE)],
                    acc_sh.at[pl.ds(r0, ROWS_PER_TILE)])
    @pl.when(s == 0)
    def _():
        pltpu.sync_copy(hsrc.at[pl.ds(c * N_NODES, NS)],
                        acc_sh.at[pl.ds(N_NODES, NS)])
    plsc.subcore_barrier()

    def chunk(j, carry):
        row = s * CHUNKS_PER_TILE + j
        pltpu.sync_copy(src2.at[row], src_v)
        pltpu.sync_copy(dst2.at[row], dst_v)
        off = c * N_NODES
        for k in range(CH // 16):
            sl = pl.ds(k * 16, 16)
            src_v[sl] = src_v[sl] + off
        pltpu.async_copy(hsrc.at[src_v], rows_v, sem).wait()
        pltpu.sync_copy(rows_v, acc_sh.at[dst_v], add=True)
        return carry

    lax.fori_loop(0, CHUNKS_PER_TILE, chunk, 0)
    plsc.subcore_barrier()

    pltpu.sync_copy(acc_sh.at[pl.ds(r0, ROWS_PER_TILE)],
                    out.at[pl.ds(c * N_NODES + r0, ROWS_PER_TILE)])


@functools.partial(jax.jit, static_argnums=())
def _propagate(hsrc, src2, dst2):
    fn = pl.kernel(
        _prop_body,
        out_type=jax.ShapeDtypeStruct((2 * N_NODES, HALF), jnp.float32),
        mesh=plsc.VectorSubcoreMesh(core_axis_name="c", subcore_axis_name="s"),
        scratch_types=[
            pltpu.VMEM_SHARED((ACC_ROWS, HALF), jnp.float32),
            pltpu.VMEM((CH,), jnp.int32),
            pltpu.VMEM((CH,), jnp.int32),
            pltpu.VMEM((CH, HALF), jnp.float32),
            pltpu.SemaphoreType.DMA,
        ],
        compiler_params=pltpu.CompilerParams(use_tc_tiling_on_sc=False),
    )
    return fn(hsrc, src2, dst2)


def _head_body(prop_ref, idx_ref, w3t_ref, b3_ref, w4t_ref, b4_ref,
               out_ref, pooled_acc):
    t = pl.program_id(0)
    c = t // 5

    @pl.when(t == 0)
    def _():
        pooled_acc[...] = jnp.zeros_like(pooled_acc)

    h = jnp.maximum(prop_ref[...], 0.0)
    idxb = idx_ref[0]
    iota = lax.broadcasted_iota(jnp.int32, (N_GRAPHS, 2000), 0)
    onehot = jnp.where(idxb == iota, 1.0, 0.0)
    part = lax.dot_general(onehot, h, (((1,), (0,)), ((), ())),
                           precision=_PREC, preferred_element_type=jnp.float32)
    csl = pl.ds(c * HALF, HALF)
    pooled_acc[:, csl] += part

    @pl.when(t == 9)
    def _():
        pooled = pooled_acc[...]
        z = lax.dot_general(pooled, w3t_ref[...], (((1,), (0,)), ((), ())),
                            precision=_PREC,
                            preferred_element_type=jnp.float32)
        z = jnp.maximum(z + b3_ref[...], 0.0)
        o = lax.dot_general(z, w4t_ref[...], (((1,), (0,)), ((), ())),
                            precision=_PREC,
                            preferred_element_type=jnp.float32)
        o = o + b4_ref[...]
        m = jnp.max(o, axis=1, keepdims=True)
        lse = m + jnp.log(jnp.sum(jnp.exp(o - m), axis=1, keepdims=True))
        out_ref[...] = o - lse


def _head(prop_flat, idx3, w3t, b3_2d, w4t, b4_2d):
    return pl.pallas_call(
        _head_body,
        grid=(10,),
        in_specs=[
            pl.BlockSpec((2000, HALF), lambda t: (t, 0)),
            pl.BlockSpec((1, 1, 2000), lambda t: (t % 5, 0, 0)),
            pl.BlockSpec((D, D), lambda t: (0, 0)),
            pl.BlockSpec((1, D), lambda t: (0, 0)),
            pl.BlockSpec((D, N_CLASS), lambda t: (0, 0)),
            pl.BlockSpec((1, N_CLASS), lambda t: (0, 0)),
        ],
        out_specs=pl.BlockSpec((N_GRAPHS, N_CLASS), lambda t: (0, 0)),
        out_shape=jax.ShapeDtypeStruct((N_GRAPHS, N_CLASS), jnp.float32),
        scratch_shapes=[pltpu.VMEM((N_GRAPHS, D), jnp.float32)],
    )(prop_flat, idx3, w3t, b3_2d, w4t, b4_2d)


def kernel(x_in, edge_index, idx, W1, b1, W2, b2, W3, b3, W4, b4):
    src = edge_index[0].astype(jnp.int32)
    dst = edge_index[1].astype(jnp.int32)
    pad = E_PAD - N_EDGES
    src2 = jnp.concatenate([src, jnp.zeros((pad,), jnp.int32)])
    src2 = src2.reshape(NS * CHUNKS_PER_TILE, CH)
    dst2 = jnp.concatenate([dst, jnp.full((pad,), N_NODES, jnp.int32)])
    dst2 = dst2.reshape(NS * CHUNKS_PER_TILE, CH)
    idx3 = idx.astype(jnp.int32).reshape(5, 1, 2000)

    h1 = _fc_in(x_in, W1.T, b1.reshape(1, D), relu_halves=False)
    p1 = _propagate(h1.reshape(2 * N_NODES, HALF), src2, dst2)
    h2 = _fc_in(p1.reshape(2, N_NODES, HALF), W2.T, b2.reshape(1, D),
                relu_halves=True)
    p2 = _propagate(h2.reshape(2 * N_NODES, HALF), src2, dst2)
    return _head(p2, idx3, W3.T, b3.reshape(1, D), W4.T,
                 b4.reshape(1, N_CLASS))
